# Initial kernel scaffold; baseline (speedup 1.0000x reference)
import functools

import jax
import jax.numpy as jnp
from jax import lax
from jax.experimental import pallas as pl
from jax.experimental.pallas import tpu as pltpu
from jax.experimental.pallas import tpu_sc as plsc

N = 10000
D = 128
HID = 100
C = 40
EDGES = 320000

F1 = 112
F2 = 48

NC = 2
NS = 16
NW = NC * NS
CHUNK = 128
CH = 80
PT = CH * CHUNK
EP = NW * PT

BN = 1000


def _mm_bias_kernel(h_ref, w_ref, b_ref, o_ref):
    o_ref[...] = jnp.dot(h_ref[...], w_ref[...],
                         preferred_element_type=jnp.float32,
                         precision=lax.Precision.HIGHEST) + b_ref[...]


def _mm_bias(h, w, b, f_out):
    n, k = h.shape
    return pl.pallas_call(
        _mm_bias_kernel,
        grid=(n // BN,),
        in_specs=[pl.BlockSpec((BN, k), lambda i: (i, 0)),
                  pl.BlockSpec((k, f_out), lambda i: (0, 0)),
                  pl.BlockSpec((1, f_out), lambda i: (0, 0))],
        out_specs=pl.BlockSpec((BN, f_out), lambda i: (i, 0)),
        out_shape=jax.ShapeDtypeStruct((n, f_out), jnp.float32),
    )(h, w, b)


def _relu_mm_bias_kernel(p_ref, w_ref, b_ref, o_ref):
    h1 = jax.nn.relu(p_ref[0] + p_ref[1])
    o_ref[...] = jnp.dot(h1, w_ref[...],
                         preferred_element_type=jnp.float32,
                         precision=lax.Precision.HIGHEST) + b_ref[...]


def _relu_mm_bias(p, w, b, f_in, f_out):
    return pl.pallas_call(
        _relu_mm_bias_kernel,
        grid=(N // BN,),
        in_specs=[pl.BlockSpec((2, BN, f_in), lambda i: (0, i, 0)),
                  pl.BlockSpec((f_in, f_out), lambda i: (0, 0)),
                  pl.BlockSpec((1, f_out), lambda i: (0, 0))],
        out_specs=pl.BlockSpec((BN, f_out), lambda i: (i, 0)),
        out_shape=jax.ShapeDtypeStruct((N, f_out), jnp.float32),
    )(p, w, b)


def _logsoftmax_kernel(q_ref, o_ref):
    x = (q_ref[0] + q_ref[1])[:, :C]
    m = jnp.max(x, axis=1, keepdims=True)
    ex = jnp.exp(x - m)
    lse = jnp.log(jnp.sum(ex, axis=1, keepdims=True))
    o_ref[...] = x - m - lse


def _logsoftmax(q):
    return pl.pallas_call(
        _logsoftmax_kernel,
        grid=(N // BN,),
        in_specs=[pl.BlockSpec((2, BN, F2), lambda i: (0, i, 0))],
        out_specs=pl.BlockSpec((BN, C), lambda i: (i, 0)),
        out_shape=jax.ShapeDtypeStruct((N, C), jnp.float32),
    )(q)


def _propagate(m_tbl, src, dst, e, f):
    nf16 = f // 16
    rz = N // NS
    mesh = plsc.VectorSubcoreMesh(core_axis_name="c", subcore_axis_name="s")

    @functools.partial(
        pl.kernel,
        out_type=jax.ShapeDtypeStruct((NC, N, f), jnp.float32),
        mesh=mesh,
        scratch_types=[
            pltpu.VMEM((PT,), jnp.int32),
            pltpu.VMEM((CH, CHUNK), jnp.int32),
            pltpu.VMEM((PT,), jnp.float32),
            pltpu.VMEM((CHUNK, f), jnp.float32),
            pltpu.VMEM((CHUNK, f), jnp.float32),
            pltpu.VMEM_SHARED((N, f), jnp.float32),
            pltpu.SemaphoreType.DMA,
        ],
    )
    def k(m_hbm, src_hbm, dst_hbm, e_hbm, out_hbm,
          src_v, dst_v, e_v, rows_v, zbuf, acc, sem):
        c = lax.axis_index("c")
        s = lax.axis_index("s")
        wid = c * NS + s
        iota = lax.iota(jnp.int32, 16)

        pltpu.sync_copy(src_hbm.at[wid], src_v)
        pltpu.sync_copy(dst_hbm.at[wid], dst_v)
        pltpu.sync_copy(e_hbm.at[wid], e_v)

        z = jnp.zeros((16,), jnp.float32)

        @pl.loop(0, CHUNK)
        def _(i):
            i16 = jnp.full((16,), i, jnp.int32)
            for t in range(nf16):
                plsc.store_scatter(zbuf, [i16, iota + t * 16], z)

        r0 = s * rz
        nfull = rz // CHUNK
        for t in range(nfull):
            pltpu.sync_copy(zbuf, acc.at[pl.ds(r0 + t * CHUNK, CHUNK)])
        rem = rz - nfull * CHUNK
        if rem:
            pltpu.sync_copy(zbuf.at[pl.ds(0, rem)],
                            acc.at[pl.ds(r0 + nfull * CHUNK, rem)])
        plsc.subcore_barrier()

        @pl.loop(0, CH)
        def _(j):
            jb = j * CHUNK
            pltpu.async_copy(m_hbm.at[src_v.at[pl.ds(jb, CHUNK)]],
                             rows_v, sem).wait()

            @pl.loop(0, CHUNK)
            def _(i):
                e_b = plsc.load_gather(e_v, [jnp.full((16,), jb + i, jnp.int32)])
                i16 = jnp.full((16,), i, jnp.int32)
                for t in range(nf16):
                    cols = iota + t * 16
                    v = plsc.load_gather(rows_v, [i16, cols])
                    plsc.store_scatter(rows_v, [i16, cols], v * e_b)

            pltpu.sync_copy(rows_v, acc.at[dst_v.at[j]], add=True)

        plsc.subcore_barrier()

        pltpu.sync_copy(acc.at[pl.ds(r0, rz)],
                        out_hbm.at[c].at[pl.ds(r0, rz)])

    return k(m_tbl, src, dst, e)


def kernel(H, A, E, W1, b1, W2, b2):
    src = A[0]
    dst = A[1]
    pad = EP - EDGES
    spread = (jnp.arange(pad, dtype=jnp.int32) * 37) % N
    srcp = jnp.concatenate([src, spread]).reshape(NW, PT)
    dstp = jnp.concatenate([dst, spread]).reshape(NW, CH, CHUNK)
    ep = jnp.concatenate([E, jnp.zeros((pad,), jnp.float32)]).reshape(NW, PT)

    w1p = jnp.zeros((D, F1), jnp.float32).at[:, :HID].set(W1)
    b1p = jnp.zeros((1, F1), jnp.float32).at[0, :HID].set(b1)
    w2p = jnp.zeros((F1, F2), jnp.float32).at[:HID, :C].set(W2)
    b2p = jnp.zeros((1, F2), jnp.float32).at[0, :C].set(b2)

    m1 = _mm_bias(H, w1p, b1p, F1)
    p = _propagate(m1, srcp, dstp, ep, F1)
    m2 = _relu_mm_bias(p, w2p, b2p, F1, F2)
    q = _propagate(m2, srcp, dstp, ep, F2)
    return _logsoftmax(q)


# speedup vs baseline: 3.6752x; 3.6752x over previous
import dataclasses
import functools

import jax
import jax.numpy as jnp
from jax import lax
from jax.experimental import pallas as pl
from jax.experimental.pallas import tpu as pltpu
from jax.experimental.pallas import tpu_sc as plsc

N = 10000
D = 128
HID = 100
C = 40
EDGES = 320000

F1 = 112
F2 = 48

NC = 2
NS = 16
NW = NC * NS
CHUNK = 128
CH = 80
PT = CH * CHUNK
EP = NW * PT

BN = 1000


def _mm_bias_kernel(h_ref, w_ref, b_ref, o_ref):
    o_ref[...] = jnp.dot(h_ref[...], w_ref[...],
                         preferred_element_type=jnp.float32,
                         precision=lax.Precision.HIGHEST) + b_ref[...]


def _mm_bias(h, w, b, f_out):
    n, k = h.shape
    return pl.pallas_call(
        _mm_bias_kernel,
        grid=(n // BN,),
        in_specs=[pl.BlockSpec((BN, k), lambda i: (i, 0)),
                  pl.BlockSpec((k, f_out), lambda i: (0, 0)),
                  pl.BlockSpec((1, f_out), lambda i: (0, 0))],
        out_specs=pl.BlockSpec((BN, f_out), lambda i: (i, 0)),
        out_shape=jax.ShapeDtypeStruct((n, f_out), jnp.float32),
    )(h, w, b)


def _relu_mm_bias_kernel(p_ref, w_ref, b_ref, o_ref):
    h1 = jax.nn.relu(p_ref[0] + p_ref[1])
    o_ref[...] = jnp.dot(h1, w_ref[...],
                         preferred_element_type=jnp.float32,
                         precision=lax.Precision.HIGHEST) + b_ref[...]


def _relu_mm_bias(p, w, b, f_in, f_out):
    return pl.pallas_call(
        _relu_mm_bias_kernel,
        grid=(N // BN,),
        in_specs=[pl.BlockSpec((2, BN, f_in), lambda i: (0, i, 0)),
                  pl.BlockSpec((f_in, f_out), lambda i: (0, 0)),
                  pl.BlockSpec((1, f_out), lambda i: (0, 0))],
        out_specs=pl.BlockSpec((BN, f_out), lambda i: (i, 0)),
        out_shape=jax.ShapeDtypeStruct((N, f_out), jnp.float32),
    )(p, w, b)


def _logsoftmax_kernel(q_ref, o_ref):
    x = (q_ref[0] + q_ref[1])[:, :C]
    m = jnp.max(x, axis=1, keepdims=True)
    ex = jnp.exp(x - m)
    lse = jnp.log(jnp.sum(ex, axis=1, keepdims=True))
    o_ref[...] = x - m - lse


def _logsoftmax(q):
    return pl.pallas_call(
        _logsoftmax_kernel,
        grid=(N // BN,),
        in_specs=[pl.BlockSpec((2, BN, F2), lambda i: (0, i, 0))],
        out_specs=pl.BlockSpec((BN, C), lambda i: (i, 0)),
        out_shape=jax.ShapeDtypeStruct((N, C), jnp.float32),
    )(q)


def _propagate(m_tbl, src, dst, e, f):
    nf16 = f // 16
    rz = 624
    tail = N - NS * rz
    mesh = plsc.VectorSubcoreMesh(core_axis_name="c", subcore_axis_name="s")
    cp = pltpu.CompilerParams()
    if "needs_layout_passes" in pltpu.CompilerParams.__dataclass_fields__:
        cp = dataclasses.replace(cp, needs_layout_passes=False)
    if "use_tc_tiling_on_sc" in pltpu.CompilerParams.__dataclass_fields__:
        cp = dataclasses.replace(cp, use_tc_tiling_on_sc=False)

    @functools.partial(
        pl.kernel,
        compiler_params=cp,
        out_type=jax.ShapeDtypeStruct((NC, N, f), jnp.float32),
        mesh=mesh,
        scratch_types=[
            pltpu.VMEM((PT,), jnp.int32),
            pltpu.VMEM((CH, CHUNK), jnp.int32),
            pltpu.VMEM((PT,), jnp.float32),
            pltpu.VMEM((CHUNK, f), jnp.float32),
            pltpu.VMEM((CHUNK, f), jnp.float32),
            pltpu.VMEM_SHARED((N, f), jnp.float32),
            pltpu.SemaphoreType.DMA,
        ],
    )
    def k(m_hbm, src_hbm, dst_hbm, e_hbm, out_hbm,
          src_v, dst_v, e_v, rows_v, zbuf, acc, sem):
        c = lax.axis_index("c")
        s = lax.axis_index("s")
        wid = c * NS + s
        iota = lax.iota(jnp.int32, 16)

        pltpu.sync_copy(src_hbm.at[wid], src_v)
        pltpu.sync_copy(dst_hbm.at[wid], dst_v)
        pltpu.sync_copy(e_hbm.at[wid], e_v)

        z = jnp.zeros((16,), jnp.float32)

        @pl.loop(0, CHUNK)
        def _(i):
            i16 = jnp.full((16,), i, jnp.int32)
            for t in range(nf16):
                plsc.store_scatter(zbuf, [i16, iota + t * 16], z)

        r0 = s * rz
        nfull = rz // CHUNK
        for t in range(nfull):
            pltpu.sync_copy(zbuf, acc.at[pl.ds(r0 + t * CHUNK, CHUNK)])
        rem = rz - nfull * CHUNK
        if rem:
            pltpu.sync_copy(zbuf.at[pl.ds(0, rem)],
                            acc.at[pl.ds(r0 + nfull * CHUNK, rem)])

        @pl.when(s == NS - 1)
        def _():
            pltpu.sync_copy(zbuf.at[pl.ds(0, tail)],
                            acc.at[pl.ds(NS * rz, tail)])

        plsc.subcore_barrier()

        @pl.loop(0, CH)
        def _(j):
            jb = j * CHUNK
            pltpu.async_copy(m_hbm.at[src_v.at[pl.ds(jb, CHUNK)]],
                             rows_v, sem).wait()

            @pl.loop(0, CHUNK)
            def _(i):
                e_b = plsc.load_gather(e_v, [jnp.full((16,), jb + i, jnp.int32)])
                i16 = jnp.full((16,), i, jnp.int32)
                for t in range(nf16):
                    cols = iota + t * 16
                    v = plsc.load_gather(rows_v, [i16, cols])
                    plsc.store_scatter(rows_v, [i16, cols], v * e_b)

            pltpu.sync_copy(rows_v, acc.at[dst_v.at[j]], add=True)

        plsc.subcore_barrier()

        pltpu.sync_copy(acc.at[pl.ds(r0, rz)],
                        out_hbm.at[c].at[pl.ds(r0, rz)])

        @pl.when(s == NS - 1)
        def _():
            pltpu.sync_copy(acc.at[pl.ds(NS * rz, tail)],
                            out_hbm.at[c].at[pl.ds(NS * rz, tail)])

    return k(m_tbl, src, dst, e)


def kernel(H, A, E, W1, b1, W2, b2):
    src = A[0]
    dst = A[1]
    pad = EP - EDGES
    spread = (jnp.arange(pad, dtype=jnp.int32) * 37) % N
    srcp = jnp.concatenate([src, spread]).reshape(NW, PT)
    dstp = jnp.concatenate([dst, spread]).reshape(NW, CH, CHUNK)
    ep = jnp.concatenate([E, jnp.zeros((pad,), jnp.float32)]).reshape(NW, PT)

    w1p = jnp.zeros((D, F1), jnp.float32).at[:, :HID].set(W1)
    b1p = jnp.zeros((1, F1), jnp.float32).at[0, :HID].set(b1)
    w2p = jnp.zeros((F1, F2), jnp.float32).at[:HID, :C].set(W2)
    b2p = jnp.zeros((1, F2), jnp.float32).at[0, :C].set(b2)

    m1 = _mm_bias(H, w1p, b1p, F1)
    p = _propagate(m1, srcp, dstp, ep, F1)
    m2 = _relu_mm_bias(p, w2p, b2p, F1, F2)
    q = _propagate(m2, srcp, dstp, ep, F2)
    return _logsoftmax(q)
